# baseline (speedup 1.0000x reference)
import functools

import jax
import jax.numpy as jnp
import numpy as np
from jax import lax
from jax.experimental import pallas as pl
from jax.experimental.pallas import tpu as pltpu
from jax.experimental.pallas import tpu_sc as plsc

N_NODES = 10000
D_FEAT = 128
N_EDGES = 320000
NUM_HOPS = 3
SIGMA = 0.1

NC = 2
NS = 16
NW = NC * NS
CHUNK = 112
NBUF = 3
NPHASE = 4
NCHUNK_P = 24
NCHUNK = NPHASE * NCHUNK_P
EDGES_PAD = NW * NCHUNK * CHUNK
PAD = EDGES_PAD - N_EDGES
ACC_ROWS = 10112
STRIPE = ACC_ROWS // NS

_ROW_BLK = 2000


def _sc_hop_body(h_hbm, src_hbm, dst_hbm, zero_hbm, out_hbm,
                 src_v, dst_v, buf0, buf1, buf2, acc, sem0, sem1, sem2):
    cid = lax.axis_index("c")
    sid = lax.axis_index("s")
    wid = sid * NC + cid
    bufs = (buf0, buf1, buf2)
    sems = (sem0, sem1, sem2)

    pltpu.sync_copy(zero_hbm, buf0)
    base = sid * STRIPE
    for k in range(STRIPE // CHUNK):
        pltpu.sync_copy(buf0, acc.at[pl.ds(base + k * CHUNK, CHUNK)])
    rem = STRIPE % CHUNK
    if rem:
        pltpu.sync_copy(buf0.at[pl.ds(0, rem)],
                        acc.at[pl.ds(base + (STRIPE // CHUNK) * CHUNK, rem)])
    plsc.subcore_barrier()

    for p in range(NPHASE):
        pltpu.sync_copy(src_hbm.at[wid, p], src_v)
        pltpu.sync_copy(dst_hbm.at[wid, p], dst_v)

        for b in range(NBUF):
            pltpu.async_copy(h_hbm.at[src_v.at[b]], bufs[b], sems[b])

        def tri(i, carry):
            j = NBUF * i
            for b in range(NBUF):
                pltpu.make_async_copy(h_hbm.at[src_v.at[j + b]],
                                      bufs[b], sems[b]).wait()
                pltpu.sync_copy(bufs[b], acc.at[dst_v.at[j + b]], add=True)
                pltpu.async_copy(h_hbm.at[src_v.at[j + b + NBUF]],
                                 bufs[b], sems[b])
            return carry

        lax.fori_loop(0, NCHUNK_P // NBUF - 1, tri, 0)
        for b in range(NBUF):
            jl = NCHUNK_P - NBUF + b
            pltpu.make_async_copy(h_hbm.at[src_v.at[jl]],
                                  bufs[b], sems[b]).wait()
            pltpu.sync_copy(bufs[b], acc.at[dst_v.at[jl]], add=True)
    plsc.subcore_barrier()

    out_base = cid * ACC_ROWS + base
    pltpu.sync_copy(acc.at[pl.ds(base, STRIPE)],
                    out_hbm.at[pl.ds(out_base, STRIPE)])


@functools.lru_cache(maxsize=None)
def _make_sc_hop(interpret: bool = False):
    mesh = plsc.VectorSubcoreMesh(core_axis_name="c", subcore_axis_name="s",
                                  num_cores=NC, num_subcores=NS)
    return functools.partial(
        pl.kernel,
        out_type=jax.ShapeDtypeStruct((NC * ACC_ROWS, D_FEAT), jnp.float32),
        mesh=mesh,
        scratch_types=[
            pltpu.VMEM((NCHUNK_P, CHUNK), jnp.int32),
            pltpu.VMEM((NCHUNK_P, CHUNK), jnp.int32),
            pltpu.VMEM((CHUNK, D_FEAT), jnp.float32),
            pltpu.VMEM((CHUNK, D_FEAT), jnp.float32),
            pltpu.VMEM((CHUNK, D_FEAT), jnp.float32),
            pltpu.VMEM_SHARED((ACC_ROWS, D_FEAT), jnp.float32),
            pltpu.SemaphoreType.DMA,
            pltpu.SemaphoreType.DMA,
            pltpu.SemaphoreType.DMA,
        ],
        interpret=interpret,
    )(_sc_hop_body)


def _norm_body(x_ref, o_ref):
    t = x_ref[...]
    ss = jnp.sum(t * t, axis=1, keepdims=True)
    o_ref[...] = t / jnp.maximum(jnp.sqrt(ss), 1e-12)


def _finish_body(p_ref, nz_ref, o_ref):
    t = p_ref[0] + p_ref[1] + nz_ref[...]
    ss = jnp.sum(t * t, axis=1, keepdims=True)
    o_ref[...] = t / jnp.maximum(jnp.sqrt(ss), 1e-12)


@functools.lru_cache(maxsize=None)
def _make_tc_kernels(interpret: bool = False):
    grid = (N_NODES // _ROW_BLK,)
    norm = pl.pallas_call(
        _norm_body,
        grid=grid,
        in_specs=[pl.BlockSpec((_ROW_BLK, D_FEAT), lambda i: (i, 0))],
        out_specs=pl.BlockSpec((_ROW_BLK, D_FEAT), lambda i: (i, 0)),
        out_shape=jax.ShapeDtypeStruct((N_NODES, D_FEAT), jnp.float32),
        interpret=interpret,
    )
    finish = pl.pallas_call(
        _finish_body,
        grid=grid,
        in_specs=[
            pl.BlockSpec((NC, _ROW_BLK, D_FEAT), lambda i: (0, i, 0)),
            pl.BlockSpec((_ROW_BLK, D_FEAT), lambda i: (i, 0)),
        ],
        out_specs=pl.BlockSpec((_ROW_BLK, D_FEAT), lambda i: (i, 0)),
        out_shape=jax.ShapeDtypeStruct((N_NODES, D_FEAT), jnp.float32),
        interpret=interpret,
    )
    return norm, finish


def _noise_const():
    key = jax.random.key(42)
    ns = []
    for _ in range(NUM_HOPS):
        key, sub = jax.random.split(key)
        ns.append(SIGMA * jax.random.normal(sub, (N_NODES, D_FEAT),
                                            dtype=jnp.float32))
    return jnp.stack(ns)


def kernel(x, edge_index):
    src = edge_index[0].astype(jnp.int32)
    dst = edge_index[1].astype(jnp.int32)
    pad_i = jnp.arange(PAD, dtype=jnp.int32)
    src_t = jnp.concatenate([src, pad_i % N_NODES]).reshape(
        NW, NPHASE, NCHUNK_P, CHUNK)
    dst_t = jnp.concatenate(
        [dst, N_NODES + pad_i % (ACC_ROWS - N_NODES)]
    ).reshape(NW, NPHASE, NCHUNK_P, CHUNK)
    zeros = jnp.zeros((CHUNK, D_FEAT), jnp.float32)
    noise = _noise_const()

    sc_hop = _make_sc_hop()
    norm, finish = _make_tc_kernels()

    h = norm(x)
    outs = [h]
    for k in range(NUM_HOPS):
        parts = sc_hop(h, src_t, dst_t, zeros)
        h = finish(parts.reshape(NC, ACC_ROWS, D_FEAT), noise[k])
        outs.append(h)
    return jnp.stack(outs)


# speedup vs baseline: 1.0191x; 1.0191x over previous
import functools

import jax
import jax.numpy as jnp
import numpy as np
from jax import lax
from jax.experimental import pallas as pl
from jax.experimental.pallas import tpu as pltpu
from jax.experimental.pallas import tpu_sc as plsc

N_NODES = 10000
D_FEAT = 128
N_EDGES = 320000
NUM_HOPS = 3
SIGMA = 0.1

NC = 2
NS = 16
NW = NC * NS
CHUNK = 112
NBUF = 3
NPHASE = 4
NCHUNK_P = 24
NCHUNK = NPHASE * NCHUNK_P
EDGES_PAD = NW * NCHUNK * CHUNK
PAD = EDGES_PAD - N_EDGES
ACC_ROWS = 10112
STRIPE = ACC_ROWS // NS

_ROW_BLK = 2000


def _sc_hop_body(h_hbm, src_hbm, dst_hbm, zero_hbm, out_hbm,
                 src_v, dst_v, buf0, buf1, buf2, acc, sem0, sem1, sem2):
    cid = lax.axis_index("c")
    sid = lax.axis_index("s")
    wid = sid * NC + cid
    bufs = (buf0, buf1, buf2)
    sems = (sem0, sem1, sem2)

    pltpu.sync_copy(zero_hbm, buf0)
    base = sid * STRIPE
    for k in range(STRIPE // CHUNK):
        pltpu.sync_copy(buf0, acc.at[pl.ds(base + k * CHUNK, CHUNK)])
    rem = STRIPE % CHUNK
    if rem:
        pltpu.sync_copy(buf0.at[pl.ds(0, rem)],
                        acc.at[pl.ds(base + (STRIPE // CHUNK) * CHUNK, rem)])
    plsc.subcore_barrier()

    for p in range(NPHASE):
        pltpu.sync_copy(src_hbm.at[wid, p], src_v)
        pltpu.sync_copy(dst_hbm.at[wid, p], dst_v)

        for b in range(NBUF):
            pltpu.async_copy(h_hbm.at[src_v.at[b]], bufs[b], sems[b])

        def tri(i, carry):
            j = NBUF * i
            for b in range(NBUF):
                pltpu.make_async_copy(h_hbm.at[src_v.at[j + b]],
                                      bufs[b], sems[b]).wait()
                pltpu.sync_copy(bufs[b], acc.at[dst_v.at[j + b]], add=True)
                pltpu.async_copy(h_hbm.at[src_v.at[j + b + NBUF]],
                                 bufs[b], sems[b])
            return carry

        lax.fori_loop(0, NCHUNK_P // NBUF - 1, tri, 0)
        for b in range(NBUF):
            jl = NCHUNK_P - NBUF + b
            pltpu.make_async_copy(h_hbm.at[src_v.at[jl]],
                                  bufs[b], sems[b]).wait()
            pltpu.sync_copy(bufs[b], acc.at[dst_v.at[jl]], add=True)
    plsc.subcore_barrier()

    out_base = cid * ACC_ROWS + base
    pltpu.sync_copy(acc.at[pl.ds(base, STRIPE)],
                    out_hbm.at[pl.ds(out_base, STRIPE)])


@functools.lru_cache(maxsize=None)
def _make_sc_hop(interpret: bool = False):
    mesh = plsc.VectorSubcoreMesh(core_axis_name="c", subcore_axis_name="s",
                                  num_cores=NC, num_subcores=NS)
    return functools.partial(
        pl.kernel,
        out_type=jax.ShapeDtypeStruct((NC * ACC_ROWS, D_FEAT), jnp.float32),
        mesh=mesh,
        scratch_types=[
            pltpu.VMEM((NCHUNK_P, CHUNK), jnp.int32),
            pltpu.VMEM((NCHUNK_P, CHUNK), jnp.int32),
            pltpu.VMEM((CHUNK, D_FEAT), jnp.float32),
            pltpu.VMEM((CHUNK, D_FEAT), jnp.float32),
            pltpu.VMEM((CHUNK, D_FEAT), jnp.float32),
            pltpu.VMEM_SHARED((ACC_ROWS, D_FEAT), jnp.float32),
            pltpu.SemaphoreType.DMA,
            pltpu.SemaphoreType.DMA,
            pltpu.SemaphoreType.DMA,
        ],
        interpret=interpret,
    )(_sc_hop_body)


def _norm_body(x_ref, o_ref):
    t = x_ref[...]
    ss = jnp.sum(t * t, axis=1, keepdims=True)
    o_ref[...] = t / jnp.maximum(jnp.sqrt(ss), 1e-12)


def _finish_body(p_ref, nz_ref, o_ref):
    t = p_ref[0] + p_ref[1] + nz_ref[...]
    ss = jnp.sum(t * t, axis=1, keepdims=True)
    o_ref[...] = t / jnp.maximum(jnp.sqrt(ss), 1e-12)


@functools.lru_cache(maxsize=None)
def _make_tc_kernels(interpret: bool = False):
    grid = (N_NODES // _ROW_BLK,)
    norm = pl.pallas_call(
        _norm_body,
        grid=grid,
        in_specs=[pl.BlockSpec((_ROW_BLK, D_FEAT), lambda i: (i, 0))],
        out_specs=pl.BlockSpec((_ROW_BLK, D_FEAT), lambda i: (i, 0)),
        out_shape=jax.ShapeDtypeStruct((N_NODES, D_FEAT), jnp.float32),
        interpret=interpret,
    )
    finish = pl.pallas_call(
        _finish_body,
        grid=grid,
        in_specs=[
            pl.BlockSpec((NC, _ROW_BLK, D_FEAT), lambda i: (0, i, 0)),
            pl.BlockSpec((_ROW_BLK, D_FEAT), lambda i: (i, 0)),
        ],
        out_specs=pl.BlockSpec((_ROW_BLK, D_FEAT), lambda i: (i, 0)),
        out_shape=jax.ShapeDtypeStruct((N_NODES, D_FEAT), jnp.float32),
        interpret=interpret,
    )
    return norm, finish


def _np_threefry2x32(k1, k2, x0, x1):
    rots = ((13, 15, 26, 6), (17, 29, 16, 24))
    ks = [np.uint32(k1), np.uint32(k2)]
    ks.append(ks[0] ^ ks[1] ^ np.uint32(0x1BD11BDA))
    x = [x0.astype(np.uint32) + ks[0], x1.astype(np.uint32) + ks[1]]
    kk = [ks[1], ks[2], ks[0]]
    rr = [rots[0], rots[1]]
    for i in range(5):
        for r in rr[0]:
            x[0] = x[0] + x[1]
            x[1] = ((x[1] << np.uint32(r)) | (x[1] >> np.uint32(32 - r)))
            x[1] = x[0] ^ x[1]
        x = [x[0] + kk[0], x[1] + kk[1] + np.uint32(i + 1)]
        kk = kk[1:] + kk[:1]
        rr = rr[1:] + rr[:1]
    return x[0], x[1]


def _np_random_bits(key, n):
    iota = np.arange(n, dtype=np.uint64)
    c1 = (iota >> np.uint64(32)).astype(np.uint32)
    c2 = iota.astype(np.uint32)
    b1, b2 = _np_threefry2x32(key[0], key[1], c1, c2)
    return b1 ^ b2


def _np_split(key):
    c1 = np.zeros(2, np.uint32)
    c2 = np.arange(2, dtype=np.uint32)
    b1, b2 = _np_threefry2x32(key[0], key[1], c1, c2)
    return (b1[0], b2[0]), (b1[1], b2[1])


def _np_ndtri(p):
    a = (-3.969683028665376e+01, 2.209460984245205e+02,
         -2.759285104469687e+02, 1.383577518672690e+02,
         -3.066479806614716e+01, 2.506628277459239e+00)
    b = (-5.447609879822406e+01, 1.615858368580409e+02,
         -1.556989798598866e+02, 6.680131188771972e+01,
         -1.328068155288572e+01)
    c = (-7.784894002430293e-03, -3.223964580411365e-01,
         -2.400758277161838e+00, -2.549732539343734e+00,
         4.374664141464968e+00, 2.938163982698783e+00)
    d = (7.784695709041462e-03, 3.224671290700398e-01,
         2.445134137142996e+00, 3.754408661907416e+00)
    p = np.asarray(p, np.float64)
    out = np.empty_like(p)
    plow, phigh = 0.02425, 1 - 0.02425
    lo = p < plow
    hi = p > phigh
    mid = ~(lo | hi)
    q = np.sqrt(-2 * np.log(p[lo]))
    out[lo] = ((((((c[0] * q + c[1]) * q + c[2]) * q + c[3]) * q + c[4]) * q
                + c[5])
               / ((((d[0] * q + d[1]) * q + d[2]) * q + d[3]) * q + 1))
    q = p[mid] - 0.5
    r = q * q
    out[mid] = ((((((a[0] * r + a[1]) * r + a[2]) * r + a[3]) * r + a[4]) * r
                 + a[5]) * q
                / (((((b[0] * r + b[1]) * r + b[2]) * r + b[3]) * r + b[4]) * r
                   + 1))
    q = np.sqrt(-2 * np.log1p(-p[hi]))
    out[hi] = -((((((c[0] * q + c[1]) * q + c[2]) * q + c[3]) * q + c[4]) * q
                 + c[5])
                / ((((d[0] * q + d[1]) * q + d[2]) * q + d[3]) * q + 1))
    return out


def _np_normal(key, n):
    bits = _np_random_bits(key, n)
    float_bits = (bits >> np.uint32(9)) | np.uint32(0x3F800000)
    floats = float_bits.view(np.float32) - np.float32(1.0)
    lo = np.float32(np.nextafter(np.float32(-1.0), np.float32(0.0)))
    hi = np.float32(1.0)
    u = np.maximum(lo, floats * (hi - lo) + lo)
    erfinv = _np_ndtri((u.astype(np.float64) + 1.0) / 2.0) / np.sqrt(2.0)
    return (np.float64(np.sqrt(2.0)) * erfinv).astype(np.float32)


@functools.lru_cache(maxsize=None)
def _noise_const():
    key = (np.uint32(0), np.uint32(42))
    ns = []
    for _ in range(NUM_HOPS):
        key, sub = _np_split(key)
        ns.append(np.float32(SIGMA)
                  * _np_normal(sub, N_NODES * D_FEAT).reshape(N_NODES, D_FEAT))
    return np.stack(ns)


def kernel(x, edge_index):
    src = edge_index[0].astype(jnp.int32)
    dst = edge_index[1].astype(jnp.int32)
    pad_i = jnp.arange(PAD, dtype=jnp.int32)
    src_t = jnp.concatenate([src, pad_i % N_NODES]).reshape(
        NW, NPHASE, NCHUNK_P, CHUNK)
    dst_t = jnp.concatenate(
        [dst, N_NODES + pad_i % (ACC_ROWS - N_NODES)]
    ).reshape(NW, NPHASE, NCHUNK_P, CHUNK)
    zeros = jnp.zeros((CHUNK, D_FEAT), jnp.float32)
    noise = jnp.asarray(_noise_const())

    sc_hop = _make_sc_hop()
    norm, finish = _make_tc_kernels()

    h = norm(x)
    outs = [h]
    for k in range(NUM_HOPS):
        parts = sc_hop(h, src_t, dst_t, zeros)
        h = finish(parts.reshape(NC, ACC_ROWS, D_FEAT), noise[k])
        outs.append(h)
    return jnp.stack(outs)


# speedup vs baseline: 1.0292x; 1.0099x over previous
import functools

import jax
import jax.numpy as jnp
import numpy as np
from jax import lax
from jax.experimental import pallas as pl
from jax.experimental.pallas import tpu as pltpu
from jax.experimental.pallas import tpu_sc as plsc

N_NODES = 10000
D_FEAT = 128
N_EDGES = 320000
NUM_HOPS = 3
SIGMA = 0.1

NC = 2
NS = 16
NW = NC * NS
CHUNK = 128
NBUF = 2
NPHASE = 2
NCHUNK_P = 40
NCHUNK = NPHASE * NCHUNK_P
EDGES_PAD = NW * NCHUNK * CHUNK
PAD = EDGES_PAD - N_EDGES
ACC_ROWS = 10112
STRIPE = ACC_ROWS // NS

_ROW_BLK = 2000


def _sc_hop_body(h_hbm, src_hbm, dst_hbm, zero_hbm, out_hbm,
                 src_v, dst_v, buf0, buf1, acc, sem0, sem1):
    cid = lax.axis_index("c")
    sid = lax.axis_index("s")
    wid = sid * NC + cid
    bufs = (buf0, buf1)
    sems = (sem0, sem1)

    pltpu.sync_copy(zero_hbm, buf0)
    base = sid * STRIPE
    for k in range(STRIPE // CHUNK):
        pltpu.sync_copy(buf0, acc.at[pl.ds(base + k * CHUNK, CHUNK)])
    rem = STRIPE % CHUNK
    if rem:
        pltpu.sync_copy(buf0.at[pl.ds(0, rem)],
                        acc.at[pl.ds(base + (STRIPE // CHUNK) * CHUNK, rem)])
    plsc.subcore_barrier()

    for p in range(NPHASE):
        pltpu.sync_copy(src_hbm.at[wid, p], src_v)
        pltpu.sync_copy(dst_hbm.at[wid, p], dst_v)

        for b in range(NBUF):
            pltpu.async_copy(h_hbm.at[src_v.at[b]], bufs[b], sems[b])

        def tri(i, carry):
            j = NBUF * i
            for b in range(NBUF):
                pltpu.make_async_copy(h_hbm.at[src_v.at[j + b]],
                                      bufs[b], sems[b]).wait()
                pltpu.sync_copy(bufs[b], acc.at[dst_v.at[j + b]], add=True)
                pltpu.async_copy(h_hbm.at[src_v.at[j + b + NBUF]],
                                 bufs[b], sems[b])
            return carry

        lax.fori_loop(0, NCHUNK_P // NBUF - 1, tri, 0)
        for b in range(NBUF):
            jl = NCHUNK_P - NBUF + b
            pltpu.make_async_copy(h_hbm.at[src_v.at[jl]],
                                  bufs[b], sems[b]).wait()
            pltpu.sync_copy(bufs[b], acc.at[dst_v.at[jl]], add=True)
    plsc.subcore_barrier()

    out_base = cid * ACC_ROWS + base
    pltpu.sync_copy(acc.at[pl.ds(base, STRIPE)],
                    out_hbm.at[pl.ds(out_base, STRIPE)])


@functools.lru_cache(maxsize=None)
def _make_sc_hop(interpret: bool = False):
    mesh = plsc.VectorSubcoreMesh(core_axis_name="c", subcore_axis_name="s",
                                  num_cores=NC, num_subcores=NS)
    return functools.partial(
        pl.kernel,
        out_type=jax.ShapeDtypeStruct((NC * ACC_ROWS, D_FEAT), jnp.float32),
        mesh=mesh,
        scratch_types=[
            pltpu.VMEM((NCHUNK_P, CHUNK), jnp.int32),
            pltpu.VMEM((NCHUNK_P, CHUNK), jnp.int32),
            pltpu.VMEM((CHUNK, D_FEAT), jnp.float32),
            pltpu.VMEM((CHUNK, D_FEAT), jnp.float32),
            pltpu.VMEM_SHARED((ACC_ROWS, D_FEAT), jnp.float32),
            pltpu.SemaphoreType.DMA,
            pltpu.SemaphoreType.DMA,
        ],
        interpret=interpret,
    )(_sc_hop_body)


def _norm_body(x_ref, o_ref):
    t = x_ref[...]
    ss = jnp.sum(t * t, axis=1, keepdims=True)
    o_ref[...] = t / jnp.maximum(jnp.sqrt(ss), 1e-12)


def _finish_body(p_ref, nz_ref, o_ref):
    t = p_ref[0] + p_ref[1] + nz_ref[...]
    ss = jnp.sum(t * t, axis=1, keepdims=True)
    o_ref[...] = t / jnp.maximum(jnp.sqrt(ss), 1e-12)


@functools.lru_cache(maxsize=None)
def _make_tc_kernels(interpret: bool = False):
    grid = (N_NODES // _ROW_BLK,)
    norm = pl.pallas_call(
        _norm_body,
        grid=grid,
        in_specs=[pl.BlockSpec((_ROW_BLK, D_FEAT), lambda i: (i, 0))],
        out_specs=pl.BlockSpec((_ROW_BLK, D_FEAT), lambda i: (i, 0)),
        out_shape=jax.ShapeDtypeStruct((N_NODES, D_FEAT), jnp.float32),
        interpret=interpret,
    )
    finish = pl.pallas_call(
        _finish_body,
        grid=grid,
        in_specs=[
            pl.BlockSpec((NC, _ROW_BLK, D_FEAT), lambda i: (0, i, 0)),
            pl.BlockSpec((_ROW_BLK, D_FEAT), lambda i: (i, 0)),
        ],
        out_specs=pl.BlockSpec((_ROW_BLK, D_FEAT), lambda i: (i, 0)),
        out_shape=jax.ShapeDtypeStruct((N_NODES, D_FEAT), jnp.float32),
        interpret=interpret,
    )
    return norm, finish


def _np_threefry2x32(k1, k2, x0, x1):
    rots = ((13, 15, 26, 6), (17, 29, 16, 24))
    ks = [np.uint32(k1), np.uint32(k2)]
    ks.append(ks[0] ^ ks[1] ^ np.uint32(0x1BD11BDA))
    x = [x0.astype(np.uint32) + ks[0], x1.astype(np.uint32) + ks[1]]
    kk = [ks[1], ks[2], ks[0]]
    rr = [rots[0], rots[1]]
    for i in range(5):
        for r in rr[0]:
            x[0] = x[0] + x[1]
            x[1] = ((x[1] << np.uint32(r)) | (x[1] >> np.uint32(32 - r)))
            x[1] = x[0] ^ x[1]
        x = [x[0] + kk[0], x[1] + kk[1] + np.uint32(i + 1)]
        kk = kk[1:] + kk[:1]
        rr = rr[1:] + rr[:1]
    return x[0], x[1]


def _np_random_bits(key, n):
    iota = np.arange(n, dtype=np.uint64)
    c1 = (iota >> np.uint64(32)).astype(np.uint32)
    c2 = iota.astype(np.uint32)
    b1, b2 = _np_threefry2x32(key[0], key[1], c1, c2)
    return b1 ^ b2


def _np_split(key):
    c1 = np.zeros(2, np.uint32)
    c2 = np.arange(2, dtype=np.uint32)
    b1, b2 = _np_threefry2x32(key[0], key[1], c1, c2)
    return (b1[0], b2[0]), (b1[1], b2[1])


def _np_ndtri(p):
    a = (-3.969683028665376e+01, 2.209460984245205e+02,
         -2.759285104469687e+02, 1.383577518672690e+02,
         -3.066479806614716e+01, 2.506628277459239e+00)
    b = (-5.447609879822406e+01, 1.615858368580409e+02,
         -1.556989798598866e+02, 6.680131188771972e+01,
         -1.328068155288572e+01)
    c = (-7.784894002430293e-03, -3.223964580411365e-01,
         -2.400758277161838e+00, -2.549732539343734e+00,
         4.374664141464968e+00, 2.938163982698783e+00)
    d = (7.784695709041462e-03, 3.224671290700398e-01,
         2.445134137142996e+00, 3.754408661907416e+00)
    p = np.asarray(p, np.float64)
    out = np.empty_like(p)
    plow, phigh = 0.02425, 1 - 0.02425
    lo = p < plow
    hi = p > phigh
    mid = ~(lo | hi)
    q = np.sqrt(-2 * np.log(p[lo]))
    out[lo] = ((((((c[0] * q + c[1]) * q + c[2]) * q + c[3]) * q + c[4]) * q
                + c[5])
               / ((((d[0] * q + d[1]) * q + d[2]) * q + d[3]) * q + 1))
    q = p[mid] - 0.5
    r = q * q
    out[mid] = ((((((a[0] * r + a[1]) * r + a[2]) * r + a[3]) * r + a[4]) * r
                 + a[5]) * q
                / (((((b[0] * r + b[1]) * r + b[2]) * r + b[3]) * r + b[4]) * r
                   + 1))
    q = np.sqrt(-2 * np.log1p(-p[hi]))
    out[hi] = -((((((c[0] * q + c[1]) * q + c[2]) * q + c[3]) * q + c[4]) * q
                 + c[5])
                / ((((d[0] * q + d[1]) * q + d[2]) * q + d[3]) * q + 1))
    return out


def _np_normal(key, n):
    bits = _np_random_bits(key, n)
    float_bits = (bits >> np.uint32(9)) | np.uint32(0x3F800000)
    floats = float_bits.view(np.float32) - np.float32(1.0)
    lo = np.float32(np.nextafter(np.float32(-1.0), np.float32(0.0)))
    hi = np.float32(1.0)
    u = np.maximum(lo, floats * (hi - lo) + lo)
    erfinv = _np_ndtri((u.astype(np.float64) + 1.0) / 2.0) / np.sqrt(2.0)
    return (np.float64(np.sqrt(2.0)) * erfinv).astype(np.float32)


@functools.lru_cache(maxsize=None)
def _noise_const():
    key = (np.uint32(0), np.uint32(42))
    ns = []
    for _ in range(NUM_HOPS):
        key, sub = _np_split(key)
        ns.append(np.float32(SIGMA)
                  * _np_normal(sub, N_NODES * D_FEAT).reshape(N_NODES, D_FEAT))
    return np.stack(ns)


def kernel(x, edge_index):
    src = edge_index[0].astype(jnp.int32)
    dst = edge_index[1].astype(jnp.int32)
    pad_i = jnp.arange(PAD, dtype=jnp.int32)
    src_t = jnp.concatenate([src, pad_i % N_NODES]).reshape(
        NW, NPHASE, NCHUNK_P, CHUNK)
    dst_t = jnp.concatenate(
        [dst, N_NODES + pad_i % (ACC_ROWS - N_NODES)]
    ).reshape(NW, NPHASE, NCHUNK_P, CHUNK)
    zeros = jnp.zeros((CHUNK, D_FEAT), jnp.float32)
    noise = jnp.asarray(_noise_const())

    sc_hop = _make_sc_hop()
    norm, finish = _make_tc_kernels()

    h = norm(x)
    outs = [h]
    for k in range(NUM_HOPS):
        parts = sc_hop(h, src_t, dst_t, zeros)
        h = finish(parts.reshape(NC, ACC_ROWS, D_FEAT), noise[k])
        outs.append(h)
    return jnp.stack(outs)


# speedup vs baseline: 1.0613x; 1.0312x over previous
import functools

import jax
import jax.numpy as jnp
import numpy as np
from jax import lax
from jax.experimental import pallas as pl
from jax.experimental.pallas import tpu as pltpu
from jax.experimental.pallas import tpu_sc as plsc

N_NODES = 10000
D_FEAT = 128
N_EDGES = 320000
NUM_HOPS = 3
SIGMA = 0.1

NC = 2
NS = 16
NW = NC * NS
CHUNK = 128
NBUF = 2
NPHASE = 2
NCHUNK_P = 40
NCHUNK = NPHASE * NCHUNK_P
EDGES_PAD = NW * NCHUNK * CHUNK
PAD = EDGES_PAD - N_EDGES
ACC_ROWS = 10112
STRIPE = ACC_ROWS // NS

_ROW_BLK = 2000


def _sc_hop_body(h_hbm, src_hbm, dst_hbm, zero_hbm, out_hbm,
                 src_v, dst_v, buf0, buf1, acc, sem0, sem1):
    cid = lax.axis_index("c")
    sid = lax.axis_index("s")
    wid = sid * NC + cid
    bufs = (buf0, buf1)
    sems = (sem0, sem1)

    pltpu.sync_copy(zero_hbm.at[wid], buf0)
    base = sid * STRIPE
    for k in range(STRIPE // CHUNK):
        pltpu.sync_copy(buf0, acc.at[pl.ds(base + k * CHUNK, CHUNK)])
    rem = STRIPE % CHUNK
    if rem:
        pltpu.sync_copy(buf0.at[pl.ds(0, rem)],
                        acc.at[pl.ds(base + (STRIPE // CHUNK) * CHUNK, rem)])
    plsc.subcore_barrier()

    for p in range(NPHASE):
        pltpu.sync_copy(src_hbm.at[wid, p], src_v)
        pltpu.sync_copy(dst_hbm.at[wid, p], dst_v)

        for b in range(NBUF):
            pltpu.async_copy(h_hbm.at[src_v.at[b]], bufs[b], sems[b])

        def tri(i, carry):
            j = NBUF * i
            for b in range(NBUF):
                pltpu.make_async_copy(h_hbm.at[src_v.at[j + b]],
                                      bufs[b], sems[b]).wait()
                pltpu.sync_copy(bufs[b], acc.at[dst_v.at[j + b]], add=True)
                pltpu.async_copy(h_hbm.at[src_v.at[j + b + NBUF]],
                                 bufs[b], sems[b])
            return carry

        lax.fori_loop(0, NCHUNK_P // NBUF - 1, tri, 0)
        for b in range(NBUF):
            jl = NCHUNK_P - NBUF + b
            pltpu.make_async_copy(h_hbm.at[src_v.at[jl]],
                                  bufs[b], sems[b]).wait()
            pltpu.sync_copy(bufs[b], acc.at[dst_v.at[jl]], add=True)
    plsc.subcore_barrier()

    out_base = cid * ACC_ROWS + base
    pltpu.sync_copy(acc.at[pl.ds(base, STRIPE)],
                    out_hbm.at[pl.ds(out_base, STRIPE)])


@functools.lru_cache(maxsize=None)
def _make_sc_hop(interpret: bool = False):
    mesh = plsc.VectorSubcoreMesh(core_axis_name="c", subcore_axis_name="s",
                                  num_cores=NC, num_subcores=NS)
    return functools.partial(
        pl.kernel,
        out_type=jax.ShapeDtypeStruct((NC * ACC_ROWS, D_FEAT), jnp.float32),
        mesh=mesh,
        scratch_types=[
            pltpu.VMEM((NCHUNK_P, CHUNK), jnp.int32),
            pltpu.VMEM((NCHUNK_P, CHUNK), jnp.int32),
            pltpu.VMEM((CHUNK, D_FEAT), jnp.float32),
            pltpu.VMEM((CHUNK, D_FEAT), jnp.float32),
            pltpu.VMEM_SHARED((ACC_ROWS, D_FEAT), jnp.float32),
            pltpu.SemaphoreType.DMA,
            pltpu.SemaphoreType.DMA,
        ],
        interpret=interpret,
    )(_sc_hop_body)


def _norm_body(x_ref, o_ref, o4_ref):
    t = x_ref[...]
    ss = jnp.sum(t * t, axis=1, keepdims=True)
    h = t / jnp.maximum(jnp.sqrt(ss), 1e-12)
    o_ref[...] = h
    o4_ref[0] = h


def _finish_body(p_ref, nz_ref, o4in_ref, o_ref, o4_ref):
    del o4in_ref
    t = p_ref[0] + p_ref[1] + nz_ref[...]
    ss = jnp.sum(t * t, axis=1, keepdims=True)
    h = t / jnp.maximum(jnp.sqrt(ss), 1e-12)
    o_ref[...] = h
    o4_ref[0] = h


@functools.lru_cache(maxsize=None)
def _make_tc_kernels(interpret: bool = False):
    grid = (N_NODES // _ROW_BLK,)
    out4_t = jax.ShapeDtypeStruct((NUM_HOPS + 1, N_NODES, D_FEAT), jnp.float32)
    h_t = jax.ShapeDtypeStruct((N_NODES, D_FEAT), jnp.float32)
    norm = pl.pallas_call(
        _norm_body,
        grid=grid,
        in_specs=[pl.BlockSpec((_ROW_BLK, D_FEAT), lambda i: (i, 0))],
        out_specs=[
            pl.BlockSpec((_ROW_BLK, D_FEAT), lambda i: (i, 0)),
            pl.BlockSpec((1, _ROW_BLK, D_FEAT), lambda i: (0, i, 0)),
        ],
        out_shape=[h_t, out4_t],
        interpret=interpret,
    )
    finishes = []
    for k in range(NUM_HOPS):
        slot = k + 1
        finishes.append(pl.pallas_call(
            _finish_body,
            grid=grid,
            in_specs=[
                pl.BlockSpec((NC, _ROW_BLK, D_FEAT), lambda i: (0, i, 0)),
                pl.BlockSpec((_ROW_BLK, D_FEAT), lambda i: (i, 0)),
                pl.BlockSpec(memory_space=pl.ANY),
            ],
            out_specs=[
                pl.BlockSpec((_ROW_BLK, D_FEAT), lambda i: (i, 0)),
                pl.BlockSpec((1, _ROW_BLK, D_FEAT),
                             lambda i, s=slot: (s, i, 0)),
            ],
            out_shape=[h_t, out4_t],
            input_output_aliases={2: 1},
            interpret=interpret,
        ))
    return norm, finishes


def _np_threefry2x32(k1, k2, x0, x1):
    rots = ((13, 15, 26, 6), (17, 29, 16, 24))
    ks = [np.uint32(k1), np.uint32(k2)]
    ks.append(ks[0] ^ ks[1] ^ np.uint32(0x1BD11BDA))
    x = [x0.astype(np.uint32) + ks[0], x1.astype(np.uint32) + ks[1]]
    kk = [ks[1], ks[2], ks[0]]
    rr = [rots[0], rots[1]]
    for i in range(5):
        for r in rr[0]:
            x[0] = x[0] + x[1]
            x[1] = ((x[1] << np.uint32(r)) | (x[1] >> np.uint32(32 - r)))
            x[1] = x[0] ^ x[1]
        x = [x[0] + kk[0], x[1] + kk[1] + np.uint32(i + 1)]
        kk = kk[1:] + kk[:1]
        rr = rr[1:] + rr[:1]
    return x[0], x[1]


def _np_random_bits(key, n):
    iota = np.arange(n, dtype=np.uint64)
    c1 = (iota >> np.uint64(32)).astype(np.uint32)
    c2 = iota.astype(np.uint32)
    b1, b2 = _np_threefry2x32(key[0], key[1], c1, c2)
    return b1 ^ b2


def _np_split(key):
    c1 = np.zeros(2, np.uint32)
    c2 = np.arange(2, dtype=np.uint32)
    b1, b2 = _np_threefry2x32(key[0], key[1], c1, c2)
    return (b1[0], b2[0]), (b1[1], b2[1])


def _np_ndtri(p):
    a = (-3.969683028665376e+01, 2.209460984245205e+02,
         -2.759285104469687e+02, 1.383577518672690e+02,
         -3.066479806614716e+01, 2.506628277459239e+00)
    b = (-5.447609879822406e+01, 1.615858368580409e+02,
         -1.556989798598866e+02, 6.680131188771972e+01,
         -1.328068155288572e+01)
    c = (-7.784894002430293e-03, -3.223964580411365e-01,
         -2.400758277161838e+00, -2.549732539343734e+00,
         4.374664141464968e+00, 2.938163982698783e+00)
    d = (7.784695709041462e-03, 3.224671290700398e-01,
         2.445134137142996e+00, 3.754408661907416e+00)
    p = np.asarray(p, np.float64)
    out = np.empty_like(p)
    plow, phigh = 0.02425, 1 - 0.02425
    lo = p < plow
    hi = p > phigh
    mid = ~(lo | hi)
    q = np.sqrt(-2 * np.log(p[lo]))
    out[lo] = ((((((c[0] * q + c[1]) * q + c[2]) * q + c[3]) * q + c[4]) * q
                + c[5])
               / ((((d[0] * q + d[1]) * q + d[2]) * q + d[3]) * q + 1))
    q = p[mid] - 0.5
    r = q * q
    out[mid] = ((((((a[0] * r + a[1]) * r + a[2]) * r + a[3]) * r + a[4]) * r
                 + a[5]) * q
                / (((((b[0] * r + b[1]) * r + b[2]) * r + b[3]) * r + b[4]) * r
                   + 1))
    q = np.sqrt(-2 * np.log1p(-p[hi]))
    out[hi] = -((((((c[0] * q + c[1]) * q + c[2]) * q + c[3]) * q + c[4]) * q
                 + c[5])
                / ((((d[0] * q + d[1]) * q + d[2]) * q + d[3]) * q + 1))
    return out


def _np_normal(key, n):
    bits = _np_random_bits(key, n)
    float_bits = (bits >> np.uint32(9)) | np.uint32(0x3F800000)
    floats = float_bits.view(np.float32) - np.float32(1.0)
    lo = np.float32(np.nextafter(np.float32(-1.0), np.float32(0.0)))
    hi = np.float32(1.0)
    u = np.maximum(lo, floats * (hi - lo) + lo)
    erfinv = _np_ndtri((u.astype(np.float64) + 1.0) / 2.0) / np.sqrt(2.0)
    return (np.float64(np.sqrt(2.0)) * erfinv).astype(np.float32)


@functools.lru_cache(maxsize=None)
def _noise_const():
    key = (np.uint32(0), np.uint32(42))
    ns = []
    for _ in range(NUM_HOPS):
        key, sub = _np_split(key)
        ns.append(np.float32(SIGMA)
                  * _np_normal(sub, N_NODES * D_FEAT).reshape(N_NODES, D_FEAT))
    return np.stack(ns)


def kernel(x, edge_index):
    src = edge_index[0].astype(jnp.int32)
    dst = edge_index[1].astype(jnp.int32)
    pad_i = jnp.arange(PAD, dtype=jnp.int32)
    src_t = jnp.concatenate([src, pad_i % N_NODES]).reshape(
        NW, NPHASE, NCHUNK_P, CHUNK)
    dst_t = jnp.concatenate(
        [dst, N_NODES + pad_i % (ACC_ROWS - N_NODES)]
    ).reshape(NW, NPHASE, NCHUNK_P, CHUNK)
    zeros = jnp.zeros((NW, CHUNK, D_FEAT), jnp.float32)
    noise = jnp.asarray(_noise_const())

    sc_hop = _make_sc_hop()
    norm, finishes = _make_tc_kernels()

    h, out4 = norm(x)
    for k in range(NUM_HOPS):
        parts = sc_hop(h, src_t, dst_t, zeros)
        h, out4 = finishes[k](parts.reshape(NC, ACC_ROWS, D_FEAT),
                              noise[k], out4)
    return out4


# speedup vs baseline: 1.0841x; 1.0215x over previous
import functools

import jax
import jax.numpy as jnp
import numpy as np
from jax import lax
from jax.experimental import pallas as pl
from jax.experimental.pallas import tpu as pltpu
from jax.experimental.pallas import tpu_sc as plsc

N_NODES = 10000
D_FEAT = 128
N_EDGES = 320000
NUM_HOPS = 3
SIGMA = 0.1

NC = 2
NS = 16
NW = NC * NS
CHUNK = 128
NBUF = 2
NPHASE = 2
NCHUNK_P = 40
NCHUNK = NPHASE * NCHUNK_P
EDGES_PAD = NW * NCHUNK * CHUNK
PAD = EDGES_PAD - N_EDGES
ACC_ROWS = 10112
STRIPE = ACC_ROWS // NS

_ROW_BLK = 5000


def _sc_hop_body(h_hbm, src_hbm, dst_hbm, zero_hbm, out_hbm,
                 src_v, dst_v, buf0, buf1, acc, sem0, sem1):
    cid = lax.axis_index("c")
    sid = lax.axis_index("s")
    wid = sid * NC + cid
    bufs = (buf0, buf1)
    sems = (sem0, sem1)

    pltpu.sync_copy(zero_hbm.at[wid], buf0)
    base = sid * STRIPE
    for k in range(STRIPE // CHUNK):
        pltpu.sync_copy(buf0, acc.at[pl.ds(base + k * CHUNK, CHUNK)])
    rem = STRIPE % CHUNK
    if rem:
        pltpu.sync_copy(buf0.at[pl.ds(0, rem)],
                        acc.at[pl.ds(base + (STRIPE // CHUNK) * CHUNK, rem)])
    plsc.subcore_barrier()

    for p in range(NPHASE):
        pltpu.sync_copy(src_hbm.at[wid, p], src_v)
        pltpu.sync_copy(dst_hbm.at[wid, p], dst_v)

        for b in range(NBUF):
            pltpu.async_copy(h_hbm.at[src_v.at[b]], bufs[b], sems[b])

        def tri(i, carry):
            j = NBUF * i
            for b in range(NBUF):
                pltpu.make_async_copy(h_hbm.at[src_v.at[j + b]],
                                      bufs[b], sems[b]).wait()
                pltpu.sync_copy(bufs[b], acc.at[dst_v.at[j + b]], add=True)
                pltpu.async_copy(h_hbm.at[src_v.at[j + b + NBUF]],
                                 bufs[b], sems[b])
            return carry

        lax.fori_loop(0, NCHUNK_P // NBUF - 1, tri, 0)
        for b in range(NBUF):
            jl = NCHUNK_P - NBUF + b
            pltpu.make_async_copy(h_hbm.at[src_v.at[jl]],
                                  bufs[b], sems[b]).wait()
            pltpu.sync_copy(bufs[b], acc.at[dst_v.at[jl]], add=True)
    plsc.subcore_barrier()

    out_base = cid * ACC_ROWS + base
    pltpu.sync_copy(acc.at[pl.ds(base, STRIPE)],
                    out_hbm.at[pl.ds(out_base, STRIPE)])


@functools.lru_cache(maxsize=None)
def _make_sc_hop(interpret: bool = False):
    mesh = plsc.VectorSubcoreMesh(core_axis_name="c", subcore_axis_name="s",
                                  num_cores=NC, num_subcores=NS)
    return functools.partial(
        pl.kernel,
        out_type=jax.ShapeDtypeStruct((NC * ACC_ROWS, D_FEAT), jnp.float32),
        mesh=mesh,
        scratch_types=[
            pltpu.VMEM((NCHUNK_P, CHUNK), jnp.int32),
            pltpu.VMEM((NCHUNK_P, CHUNK), jnp.int32),
            pltpu.VMEM((CHUNK, D_FEAT), jnp.float32),
            pltpu.VMEM((CHUNK, D_FEAT), jnp.float32),
            pltpu.VMEM_SHARED((ACC_ROWS, D_FEAT), jnp.float32),
            pltpu.SemaphoreType.DMA,
            pltpu.SemaphoreType.DMA,
        ],
        interpret=interpret,
    )(_sc_hop_body)


def _norm_body(x_ref, o_ref, o4_ref):
    t = x_ref[...]
    ss = jnp.sum(t * t, axis=1, keepdims=True)
    h = t / jnp.maximum(jnp.sqrt(ss), 1e-12)
    o_ref[...] = h
    o4_ref[0] = h


def _finish_body(p_ref, nz_ref, o4in_ref, o_ref, o4_ref):
    del o4in_ref
    t = p_ref[0] + p_ref[1] + nz_ref[...]
    ss = jnp.sum(t * t, axis=1, keepdims=True)
    h = t / jnp.maximum(jnp.sqrt(ss), 1e-12)
    o_ref[...] = h
    o4_ref[0] = h


@functools.lru_cache(maxsize=None)
def _make_tc_kernels(interpret: bool = False):
    grid = (N_NODES // _ROW_BLK,)
    out4_t = jax.ShapeDtypeStruct((NUM_HOPS + 1, N_NODES, D_FEAT), jnp.float32)
    h_t = jax.ShapeDtypeStruct((N_NODES, D_FEAT), jnp.float32)
    norm = pl.pallas_call(
        _norm_body,
        grid=grid,
        in_specs=[pl.BlockSpec((_ROW_BLK, D_FEAT), lambda i: (i, 0))],
        out_specs=[
            pl.BlockSpec((_ROW_BLK, D_FEAT), lambda i: (i, 0)),
            pl.BlockSpec((1, _ROW_BLK, D_FEAT), lambda i: (0, i, 0)),
        ],
        out_shape=[h_t, out4_t],
        interpret=interpret,
    )
    finishes = []
    for k in range(NUM_HOPS):
        slot = k + 1
        finishes.append(pl.pallas_call(
            _finish_body,
            grid=grid,
            in_specs=[
                pl.BlockSpec((NC, _ROW_BLK, D_FEAT), lambda i: (0, i, 0)),
                pl.BlockSpec((_ROW_BLK, D_FEAT), lambda i: (i, 0)),
                pl.BlockSpec(memory_space=pl.ANY),
            ],
            out_specs=[
                pl.BlockSpec((_ROW_BLK, D_FEAT), lambda i: (i, 0)),
                pl.BlockSpec((1, _ROW_BLK, D_FEAT),
                             lambda i, s=slot: (s, i, 0)),
            ],
            out_shape=[h_t, out4_t],
            input_output_aliases={2: 1},
            interpret=interpret,
        ))
    return norm, finishes


def _np_threefry2x32(k1, k2, x0, x1):
    rots = ((13, 15, 26, 6), (17, 29, 16, 24))
    ks = [np.uint32(k1), np.uint32(k2)]
    ks.append(ks[0] ^ ks[1] ^ np.uint32(0x1BD11BDA))
    x = [x0.astype(np.uint32) + ks[0], x1.astype(np.uint32) + ks[1]]
    kk = [ks[1], ks[2], ks[0]]
    rr = [rots[0], rots[1]]
    for i in range(5):
        for r in rr[0]:
            x[0] = x[0] + x[1]
            x[1] = ((x[1] << np.uint32(r)) | (x[1] >> np.uint32(32 - r)))
            x[1] = x[0] ^ x[1]
        x = [x[0] + kk[0], x[1] + kk[1] + np.uint32(i + 1)]
        kk = kk[1:] + kk[:1]
        rr = rr[1:] + rr[:1]
    return x[0], x[1]


def _np_random_bits(key, n):
    iota = np.arange(n, dtype=np.uint64)
    c1 = (iota >> np.uint64(32)).astype(np.uint32)
    c2 = iota.astype(np.uint32)
    b1, b2 = _np_threefry2x32(key[0], key[1], c1, c2)
    return b1 ^ b2


def _np_split(key):
    c1 = np.zeros(2, np.uint32)
    c2 = np.arange(2, dtype=np.uint32)
    b1, b2 = _np_threefry2x32(key[0], key[1], c1, c2)
    return (b1[0], b2[0]), (b1[1], b2[1])


def _np_ndtri(p):
    a = (-3.969683028665376e+01, 2.209460984245205e+02,
         -2.759285104469687e+02, 1.383577518672690e+02,
         -3.066479806614716e+01, 2.506628277459239e+00)
    b = (-5.447609879822406e+01, 1.615858368580409e+02,
         -1.556989798598866e+02, 6.680131188771972e+01,
         -1.328068155288572e+01)
    c = (-7.784894002430293e-03, -3.223964580411365e-01,
         -2.400758277161838e+00, -2.549732539343734e+00,
         4.374664141464968e+00, 2.938163982698783e+00)
    d = (7.784695709041462e-03, 3.224671290700398e-01,
         2.445134137142996e+00, 3.754408661907416e+00)
    p = np.asarray(p, np.float64)
    out = np.empty_like(p)
    plow, phigh = 0.02425, 1 - 0.02425
    lo = p < plow
    hi = p > phigh
    mid = ~(lo | hi)
    q = np.sqrt(-2 * np.log(p[lo]))
    out[lo] = ((((((c[0] * q + c[1]) * q + c[2]) * q + c[3]) * q + c[4]) * q
                + c[5])
               / ((((d[0] * q + d[1]) * q + d[2]) * q + d[3]) * q + 1))
    q = p[mid] - 0.5
    r = q * q
    out[mid] = ((((((a[0] * r + a[1]) * r + a[2]) * r + a[3]) * r + a[4]) * r
                 + a[5]) * q
                / (((((b[0] * r + b[1]) * r + b[2]) * r + b[3]) * r + b[4]) * r
                   + 1))
    q = np.sqrt(-2 * np.log1p(-p[hi]))
    out[hi] = -((((((c[0] * q + c[1]) * q + c[2]) * q + c[3]) * q + c[4]) * q
                 + c[5])
                / ((((d[0] * q + d[1]) * q + d[2]) * q + d[3]) * q + 1))
    return out


def _np_normal(key, n):
    bits = _np_random_bits(key, n)
    float_bits = (bits >> np.uint32(9)) | np.uint32(0x3F800000)
    floats = float_bits.view(np.float32) - np.float32(1.0)
    lo = np.float32(np.nextafter(np.float32(-1.0), np.float32(0.0)))
    hi = np.float32(1.0)
    u = np.maximum(lo, floats * (hi - lo) + lo)
    erfinv = _np_ndtri((u.astype(np.float64) + 1.0) / 2.0) / np.sqrt(2.0)
    return (np.float64(np.sqrt(2.0)) * erfinv).astype(np.float32)


@functools.lru_cache(maxsize=None)
def _noise_const():
    key = (np.uint32(0), np.uint32(42))
    ns = []
    for _ in range(NUM_HOPS):
        key, sub = _np_split(key)
        ns.append(np.float32(SIGMA)
                  * _np_normal(sub, N_NODES * D_FEAT).reshape(N_NODES, D_FEAT))
    return np.stack(ns)


def kernel(x, edge_index):
    src = edge_index[0].astype(jnp.int32)
    dst = edge_index[1].astype(jnp.int32)
    pad_i = jnp.arange(PAD, dtype=jnp.int32)
    src_t = jnp.concatenate([src, pad_i % N_NODES]).reshape(
        NW, NPHASE, NCHUNK_P, CHUNK)
    dst_t = jnp.concatenate(
        [dst, N_NODES + pad_i % (ACC_ROWS - N_NODES)]
    ).reshape(NW, NPHASE, NCHUNK_P, CHUNK)
    zeros = jnp.zeros((NW, CHUNK, D_FEAT), jnp.float32)
    noise = jnp.asarray(_noise_const())

    sc_hop = _make_sc_hop()
    norm, finishes = _make_tc_kernels()

    h, out4 = norm(x)
    for k in range(NUM_HOPS):
        parts = sc_hop(h, src_t, dst_t, zeros)
        h, out4 = finishes[k](parts.reshape(NC, ACC_ROWS, D_FEAT),
                              noise[k], out4)
    return out4
